# baseline (speedup 1.0000x reference)
import jax
import jax.numpy as jnp
from jax import lax
from jax.experimental import pallas as pl
from jax.experimental.pallas import tpu as pltpu
from jax.experimental.pallas import tpu_sc as plsc

B = 1024
D = 64
LENS = (20, 50, 100, 200)
NC, NS = 2, 16
LC = 16
NBUF = 2

SC_INPUTS = (0,)
TC_INPUTS = (1, 2, 3)
TC_GRID = 10
TC_LCS = tuple(LENS[i] // TC_GRID for i in TC_INPUTS)

CHUNKS = []
for _slot, _i in enumerate(SC_INPUTS):
    _L = LENS[_i]
    _l0 = 0
    while _l0 < _L:
        _lc = min(LC, _L - _l0)
        CHUNKS.append((_slot, _l0, _lc, _l0 == 0, _l0 + _lc == _L))
        _l0 += _lc


def _unroll_for(lc):
    for u in (4, 2):
        if lc % u == 0:
            return u
    return 1


def _sc_body(*refs):
    n_in = len(SC_INPUTS)
    ins = refs[:n_in]
    out = refs[n_in]
    scratch = refs[n_in + 1:]
    bufs = scratch[:NBUF]
    acc = scratch[NBUF]
    sems = scratch[NBUF + 1:]
    w = lax.axis_index("s") * NC + lax.axis_index("c")
    dg = w // 4
    bp = w % 4
    d0 = dg * 8
    b0 = bp * 256

    def issue(j):
        slot, l0, lc, _, _ = CHUNKS[j]
        return pltpu.async_copy(
            ins[slot].at[pl.ds(l0, lc), pl.ds(d0, 8), pl.ds(b0, 256)],
            bufs[j % NBUF].at[pl.ds(0, lc)],
            sems[j % NBUF],
        )

    n = len(CHUNKS)
    descs = [None] * n
    for j in range(min(NBUF, n)):
        descs[j] = issue(j)
    for j, (slot, l0, lc, first, last) in enumerate(CHUNKS):
        descs[j].wait()
        buf = bufs[j % NBUF]
        unroll = _unroll_for(lc)

        def col_body(c, carry):
            s = c // 16
            koff = (c % 16) * 16
            zero = jnp.zeros((16,), jnp.float32)

            @plsc.parallel_loop(0, lc, step=1, unroll=unroll, carry=zero)
            def colsum(l, a):
                return a + buf[l, s, pl.ds(koff, 16)]

            if first:
                acc[s, pl.ds(koff, 16)] = colsum
            else:
                acc[s, pl.ds(koff, 16)] = acc[s, pl.ds(koff, 16)] + colsum
            return carry

        lax.fori_loop(0, 128, col_body, jnp.int32(0))
        if last:
            pltpu.sync_copy(acc, out.at[slot, dg, bp])
        if j + NBUF < n:
            descs[j + NBUF] = issue(j + NBUF)


def _build_sc_call():
    mesh = plsc.VectorSubcoreMesh(
        core_axis_name="c", subcore_axis_name="s", num_cores=NC, num_subcores=NS
    )
    scratch = [pltpu.VMEM((LC, 8, 256), jnp.float32) for _ in range(NBUF)]
    scratch += [pltpu.VMEM((8, 256), jnp.float32)]
    scratch += [pltpu.SemaphoreType.DMA for _ in range(NBUF)]
    return pl.kernel(
        _sc_body,
        out_type=jax.ShapeDtypeStruct((len(SC_INPUTS), 8, 4, 8, 256), jnp.float32),
        mesh=mesh,
        scratch_types=scratch,
        compiler_params=pltpu.CompilerParams(use_tc_tiling_on_sc=True),
    )


def _tc_body(x1, x2, x3, o_ref):
    g = pl.program_id(0)

    @pl.when(g == 0)
    def _():
        o_ref[...] = jnp.zeros_like(o_ref)

    for i, x in enumerate((x1, x2, x3)):
        o_ref[i, :, :] += jnp.sum(x[...], axis=0)


def kernel(inputs_0, inputs_1, inputs_2, inputs_3, sum_dim, concat_mode,
           keep_dims, cat_axis, is_cat):
    xs = (inputs_0, inputs_1, inputs_2, inputs_3)
    xt = [jnp.transpose(t, (1, 2, 0)) for t in xs]

    sc_out = _build_sc_call()(*[xt[i] for i in SC_INPUTS])
    sc_part = sc_out.transpose(2, 4, 0, 1, 3).reshape(B, len(SC_INPUTS), D)

    tc_out = pl.pallas_call(
        _tc_body,
        grid=(TC_GRID,),
        in_specs=[
            pl.BlockSpec((lc, D, B), lambda g, _lc=lc: (g, 0, 0))
            for lc in TC_LCS
        ],
        out_specs=pl.BlockSpec((3, D, B), lambda g: (0, 0, 0)),
        out_shape=jax.ShapeDtypeStruct((3, D, B), jnp.float32),
        compiler_params=pltpu.CompilerParams(
            dimension_semantics=("arbitrary",),
        ),
    )(*[xt[i] for i in TC_INPUTS])
    tc_part = tc_out.transpose(2, 0, 1)

    return jnp.concatenate([sc_part, tc_part], axis=1)


# speedup vs baseline: 1.5612x; 1.5612x over previous
import jax
import jax.numpy as jnp
from jax.experimental import pallas as pl
from jax.experimental.pallas import tpu as pltpu

B = 1024
D = 64
LENS = (20, 50, 100, 200)
GRID = 5
LCS = tuple(L // GRID for L in LENS)


def _tc_body(x0, x1, x2, x3, o_ref):
    g = pl.program_id(0)

    @pl.when(g == 0)
    def _():
        o_ref[...] = jnp.zeros_like(o_ref)

    for i, x in enumerate((x0, x1, x2, x3)):
        o_ref[i, :, :] += jnp.sum(x[...], axis=0)


def kernel(inputs_0, inputs_1, inputs_2, inputs_3, sum_dim, concat_mode,
           keep_dims, cat_axis, is_cat):
    xs = (inputs_0, inputs_1, inputs_2, inputs_3)
    xt = [jnp.transpose(t, (1, 2, 0)) for t in xs]
    out = pl.pallas_call(
        _tc_body,
        grid=(GRID,),
        in_specs=[
            pl.BlockSpec((lc, D, B), lambda g, _lc=lc: (g, 0, 0))
            for lc in LCS
        ],
        out_specs=pl.BlockSpec((4, D, B), lambda g: (0, 0, 0)),
        out_shape=jax.ShapeDtypeStruct((4, D, B), jnp.float32),
        compiler_params=pltpu.CompilerParams(
            dimension_semantics=("arbitrary",),
        ),
    )(*xt)
    return out.transpose(2, 0, 1)


# speedup vs baseline: 1.5874x; 1.0168x over previous
import jax
import jax.numpy as jnp
from jax.experimental import pallas as pl
from jax.experimental.pallas import tpu as pltpu

B = 1024
D = 64
LENS = (20, 50, 100, 200)
GRID = 10
LCS = tuple(L // GRID for L in LENS)


def _tc_body(x0, x1, x2, x3, o_ref):
    g = pl.program_id(0)

    @pl.when(g == 0)
    def _():
        o_ref[...] = jnp.zeros_like(o_ref)

    for i, x in enumerate((x0, x1, x2, x3)):
        o_ref[i, :, :] += jnp.sum(x[...], axis=0)


def kernel(inputs_0, inputs_1, inputs_2, inputs_3, sum_dim, concat_mode,
           keep_dims, cat_axis, is_cat):
    xs = (inputs_0, inputs_1, inputs_2, inputs_3)
    xt = [jnp.transpose(t, (1, 2, 0)) for t in xs]
    out = pl.pallas_call(
        _tc_body,
        grid=(GRID,),
        in_specs=[
            pl.BlockSpec((lc, D, B), lambda g, _lc=lc: (g, 0, 0))
            for lc in LCS
        ],
        out_specs=pl.BlockSpec((4, D, B), lambda g: (0, 0, 0)),
        out_shape=jax.ShapeDtypeStruct((4, D, B), jnp.float32),
        compiler_params=pltpu.CompilerParams(
            dimension_semantics=("arbitrary",),
        ),
    )(*xt)
    return out.transpose(2, 0, 1)
